# Initial kernel scaffold; baseline (speedup 1.0000x reference)
import jax
import jax.numpy as jnp
from jax.experimental import pallas as pl


def kernel(node_feats, cond_feats, weights, params, coords_j, W_cond, b_cond, W_film, b_film):
    raise NotImplementedError("write your pallas kernel here")


# speedup vs baseline: 1.5764x; 1.5764x over previous
import functools
import math

import jax
import jax.numpy as jnp
from jax import lax
from jax.experimental import pallas as pl
from jax.experimental.pallas import tpu as pltpu
from jax.experimental.pallas import tpu_sc as plsc

_NC = 2
_NS = 16
_NW = _NC * _NS
_LANES = 16


def _film_body(od, nf_ref, cond_ref, w_ref, p_ref, Wc_ref, bc_ref, Wf_ref,
               bf_ref, tbl_ref, ew_ref):
    nf = nf_ref[0]
    x = lax.dot_general(nf, Wf_ref[...], (((1,), (1,)), ((), ())),
                        preferred_element_type=jnp.float32)
    x = x + bf_ref[...]
    mu = jnp.mean(x, axis=1, keepdims=True)
    xc = x - mu
    var = jnp.mean(xc * xc, axis=1, keepdims=True)
    xn = xc / jnp.sqrt(var + 1e-5)
    gb = lax.dot_general(cond_ref[0], Wc_ref[...], (((1,), (1,)), ((), ())),
                         preferred_element_type=jnp.float32)
    gb = gb + bc_ref[...]
    gamma = gb[:, :od] + 1.0
    beta = gb[:, od:]
    tbl_ref[0] = jnp.maximum(gamma * xn + beta, 0.0)
    ew_ref[0] = w_ref[0] * p_ref[0]


def _film_call(node_feats, cond_feats, w2, p2, W_cond, b_cond, W_film, b_film):
    B, N, D = node_feats.shape
    OD = W_film.shape[0]
    CD = W_cond.shape[1]
    R = w2.shape[1]
    grid = (B,)
    return pl.pallas_call(
        functools.partial(_film_body, OD),
        grid=grid,
        in_specs=[
            pl.BlockSpec((1, N, D), lambda b: (b, 0, 0)),
            pl.BlockSpec((1, 1, CD), lambda b: (b, 0, 0)),
            pl.BlockSpec((1, R, 128), lambda b: (b, 0, 0)),
            pl.BlockSpec((1, R, 128), lambda b: (b, 0, 0)),
            pl.BlockSpec((2 * OD, CD), lambda b: (0, 0)),
            pl.BlockSpec((1, 2 * OD), lambda b: (0, 0)),
            pl.BlockSpec((OD, D), lambda b: (0, 0)),
            pl.BlockSpec((1, OD), lambda b: (0, 0)),
        ],
        out_specs=[
            pl.BlockSpec((1, N, OD), lambda b: (b, 0, 0)),
            pl.BlockSpec((1, R, 128), lambda b: (b, 0, 0)),
        ],
        out_shape=[
            jax.ShapeDtypeStruct((B, N, OD), jnp.float32),
            jax.ShapeDtypeStruct((B, R, 128), jnp.float32),
        ],
    )(node_feats, cond_feats, w2, p2, W_cond, b_cond.reshape(1, 2 * OD),
      W_film, b_film.reshape(1, OD))


def _make_sc_gather(n_nodes_tbl, OD, K, CPW, CE):
    CN = CE // K
    RPW = CPW * CN
    EPW = CPW * CE
    NCH = OD // _LANES
    mesh = plsc.VectorSubcoreMesh(core_axis_name="c", subcore_axis_name="s")

    @functools.partial(
        pl.kernel,
        out_type=jax.ShapeDtypeStruct((_NW * RPW, OD), jnp.float32),
        mesh=mesh,
        scratch_types=[
            pltpu.VMEM((EPW,), jnp.int32),
            pltpu.VMEM((EPW,), jnp.float32),
            pltpu.VMEM((2, CE, OD), jnp.float32),
            pltpu.VMEM((RPW, OD), jnp.float32),
            pltpu.SemaphoreType.DMA,
            pltpu.SemaphoreType.DMA,
        ],
    )
    def sc_gather(tbl_hbm, idx_hbm, ew_hbm, out_hbm, idx_v, ew_v, rows_v,
                  out_v, sem0, sem1):
        wid = lax.axis_index("s") * _NC + lax.axis_index("c")
        ebase = wid * EPW
        pltpu.sync_copy(idx_hbm.at[pl.ds(ebase, EPW)], idx_v)
        pltpu.sync_copy(ew_hbm.at[pl.ds(ebase, EPW)], ew_v)
        pltpu.async_copy(tbl_hbm.at[idx_v.at[pl.ds(0, CE)]], rows_v.at[0],
                         sem0)

        lane_splat = [jnp.full((_LANES,), j, jnp.int32) for j in range(_LANES)]

        def do_chunk(ci, slot, sem):
            pltpu.make_async_copy(tbl_hbm.at[idx_v.at[pl.ds(0, CE)]],
                                  rows_v.at[slot], sem).wait()
            for q in range(CN):
                acc = [jnp.zeros((_LANES,), jnp.float32) for _ in range(NCH)]
                for g in range(K // _LANES):
                    wv = ew_v[pl.ds(ci * CE + q * K + g * _LANES, _LANES)]
                    for jj in range(_LANES):
                        e = q * K + g * _LANES + jj
                        wb = wv.at[lane_splat[jj]].get(
                            mode='promise_in_bounds')
                        for c in range(NCH):
                            r = rows_v[slot, e, pl.ds(c * _LANES, _LANES)]
                            acc[c] = acc[c] + wb * r
                row = ci * CN + q
                for c in range(NCH):
                    out_v[row, pl.ds(c * _LANES, _LANES)] = jnp.maximum(
                        acc[c], 0.0)

        def gbody(g, carry):
            pltpu.async_copy(
                tbl_hbm.at[idx_v.at[pl.ds((2 * g + 1) * CE, CE)]],
                rows_v.at[1], sem1)
            do_chunk(2 * g, 0, sem0)

            @pl.when(2 * g + 2 < CPW)
            def _():
                pltpu.async_copy(
                    tbl_hbm.at[idx_v.at[pl.ds((2 * g + 2) * CE, CE)]],
                    rows_v.at[0], sem0)

            do_chunk(2 * g + 1, 1, sem1)
            return carry

        lax.fori_loop(0, CPW // 2, gbody, 0)
        pltpu.sync_copy(out_v, out_hbm.at[pl.ds(wid * RPW, RPW)])

    return sc_gather


def kernel(node_feats, cond_feats, weights, params, coords_j, W_cond, b_cond,
           W_film, b_film):
    B, N, D = node_feats.shape
    K = weights.shape[2]
    OD = W_film.shape[0]
    E = B * N * K

    w2 = weights.reshape(B, (N * K) // 128, 128)
    p2 = params.reshape(B, (N * K) // 128, 128)
    tbl, ew = _film_call(node_feats, cond_feats, w2, p2, W_cond, b_cond,
                         W_film, b_film)
    tbl_flat = tbl.reshape(B * N, OD)
    ew_flat = ew.reshape(E)

    CE = 128
    n_chunks = E // CE
    CPW = -(-n_chunks // _NW)
    CPW += CPW % 2
    pad_E = _NW * CPW * CE
    idx = jnp.pad(coords_j.astype(jnp.int32), (0, pad_E - E))
    eww = jnp.pad(ew_flat, (0, pad_E - E))

    sc = _make_sc_gather(B * N, OD, K, CPW, CE)
    out = sc(tbl_flat, idx, eww)
    return out[:B * N].reshape(B, N, OD)


# speedup vs baseline: 5.7057x; 3.6195x over previous
import functools

import jax
import jax.numpy as jnp
from jax import lax
from jax.experimental import pallas as pl
from jax.experimental.pallas import tpu as pltpu
from jax.experimental.pallas import tpu_sc as plsc

_NC = 2
_NS = 16
_NW = _NC * _NS
_LANES = 16


def _film_body(od, nf_ref, cond_ref, Wc_ref, bc_ref, Wf_ref, bf_ref, tbl_ref):
    nf = nf_ref[0]
    x = lax.dot_general(nf, Wf_ref[...], (((1,), (1,)), ((), ())),
                        preferred_element_type=jnp.float32)
    x = x + bf_ref[...]
    mu = jnp.mean(x, axis=1, keepdims=True)
    xc = x - mu
    var = jnp.mean(xc * xc, axis=1, keepdims=True)
    xn = xc / jnp.sqrt(var + 1e-5)
    gb = lax.dot_general(cond_ref[0], Wc_ref[...], (((1,), (1,)), ((), ())),
                         preferred_element_type=jnp.float32)
    gb = gb + bc_ref[...]
    gamma = gb[:, :od] + 1.0
    beta = gb[:, od:]
    tbl_ref[0] = jnp.maximum(gamma * xn + beta, 0.0)


def _film_call(node_feats, cond_feats, W_cond, b_cond, W_film, b_film):
    B, N, D = node_feats.shape
    OD = W_film.shape[0]
    CD = W_cond.shape[1]
    return pl.pallas_call(
        functools.partial(_film_body, OD),
        grid=(B,),
        in_specs=[
            pl.BlockSpec((1, N, D), lambda b: (b, 0, 0)),
            pl.BlockSpec((1, 1, CD), lambda b: (b, 0, 0)),
            pl.BlockSpec((2 * OD, CD), lambda b: (0, 0)),
            pl.BlockSpec((1, 2 * OD), lambda b: (0, 0)),
            pl.BlockSpec((OD, D), lambda b: (0, 0)),
            pl.BlockSpec((1, OD), lambda b: (0, 0)),
        ],
        out_specs=pl.BlockSpec((1, N, OD), lambda b: (b, 0, 0)),
        out_shape=jax.ShapeDtypeStruct((B, N, OD), jnp.float32),
    )(node_feats, cond_feats, W_cond, b_cond.reshape(1, 2 * OD), W_film,
      b_film.reshape(1, OD))


def _make_sc_gather(n_nodes, OD, K, E):
    CE = 128
    CN = CE // K
    NCH = OD // _LANES
    n_chunks = E // CE
    assert n_chunks * CE == E
    BASE = n_chunks // _NW
    assert BASE % 3 == 0
    assert BASE % 2 == 0
    REM = n_chunks - _NW * BASE
    assert REM % 2 == 0
    NXW = REM // 2
    CAP = BASE + (2 if REM else 0)
    mesh = plsc.VectorSubcoreMesh(core_axis_name="c", subcore_axis_name="s")

    @functools.partial(
        pl.kernel,
        out_type=jax.ShapeDtypeStruct((n_nodes, OD), jnp.float32),
        mesh=mesh,
        scratch_types=[
            pltpu.VMEM((CAP * CE,), jnp.int32),
            pltpu.VMEM((CAP * CE,), jnp.float32),
            pltpu.VMEM((CAP * CE,), jnp.float32),
            pltpu.VMEM((3, CE, OD), jnp.float32),
            pltpu.VMEM((CAP * CN, OD), jnp.float32),
            pltpu.SemaphoreType.DMA((3,)),
        ],
    )
    def sc_gather(tbl_hbm, idx_hbm, w_hbm, p_hbm, out_hbm, idx_v, w_v, p_v,
                  rows_v, out_v, sem):
        wid = lax.axis_index("s") * _NC + lax.axis_index("c")
        start = wid * BASE + 2 * jnp.minimum(wid, NXW)
        ebase = start * CE
        has_extra = wid < NXW
        pltpu.sync_copy(idx_hbm.at[pl.ds(ebase, BASE * CE)],
                        idx_v.at[pl.ds(0, BASE * CE)])
        pltpu.sync_copy(w_hbm.at[pl.ds(ebase, BASE * CE)],
                        w_v.at[pl.ds(0, BASE * CE)])
        pltpu.sync_copy(p_hbm.at[pl.ds(ebase, BASE * CE)],
                        p_v.at[pl.ds(0, BASE * CE)])

        @pl.when(has_extra)
        def _():
            eb2 = ebase + BASE * CE
            pltpu.sync_copy(idx_hbm.at[pl.ds(eb2, 2 * CE)],
                            idx_v.at[pl.ds(BASE * CE, 2 * CE)])
            pltpu.sync_copy(w_hbm.at[pl.ds(eb2, 2 * CE)],
                            w_v.at[pl.ds(BASE * CE, 2 * CE)])
            pltpu.sync_copy(p_hbm.at[pl.ds(eb2, 2 * CE)],
                            p_v.at[pl.ds(BASE * CE, 2 * CE)])

        lane_splat = [jnp.full((_LANES,), j, jnp.int32) for j in range(_LANES)]

        def issue(ci, slot):
            pltpu.async_copy(tbl_hbm.at[idx_v.at[pl.ds(ci * CE, CE)]],
                             rows_v.at[slot], sem.at[slot])

        def wait(slot):
            pltpu.make_async_copy(tbl_hbm.at[idx_v.at[pl.ds(0, CE)]],
                                  rows_v.at[slot], sem.at[slot]).wait()

        def compute(ci, slot):
            for q in range(CN):
                acc = [jnp.zeros((_LANES,), jnp.float32) for _ in range(NCH)]
                for g in range(K // _LANES):
                    off = ci * CE + q * K + g * _LANES
                    ew = (w_v[pl.ds(off, _LANES)] * p_v[pl.ds(off, _LANES)])
                    for jj in range(_LANES):
                        e = q * K + g * _LANES + jj
                        wb = ew.at[lane_splat[jj]].get(
                            mode='promise_in_bounds')
                        for c in range(NCH):
                            r = rows_v[slot, e, pl.ds(c * _LANES, _LANES)]
                            acc[c] = acc[c] + wb * r
                row = ci * CN + q
                for c in range(NCH):
                    out_v[row, pl.ds(c * _LANES, _LANES)] = jnp.maximum(
                        acc[c], 0.0)

        n_mine = BASE + jnp.where(has_extra, 2, 0)
        issue(0, 0)
        issue(1, 1)

        def gbody(i, carry):
            slot = lax.rem(i, 3)
            wait(slot)
            compute(i, slot)
            nci = i + 2

            @pl.when(nci < n_mine)
            def _():
                issue(nci, lax.rem(nci, 3))
            return carry

        lax.fori_loop(0, n_mine, gbody, 0)

        pltpu.sync_copy(out_v.at[pl.ds(0, BASE * CN)],
                        out_hbm.at[pl.ds(start * CN, BASE * CN)])

        @pl.when(has_extra)
        def _():
            pltpu.sync_copy(
                out_v.at[pl.ds(BASE * CN, 2 * CN)],
                out_hbm.at[pl.ds(start * CN + BASE * CN, 2 * CN)])

    return sc_gather


def kernel(node_feats, cond_feats, weights, params, coords_j, W_cond, b_cond,
           W_film, b_film):
    B, N, D = node_feats.shape
    K = weights.shape[2]
    OD = W_film.shape[0]
    E = B * N * K

    tbl = _film_call(node_feats, cond_feats, W_cond, b_cond, W_film, b_film)
    sc = _make_sc_gather(B * N, OD, K, E)
    out = sc(tbl.reshape(B * N, OD), coords_j.astype(jnp.int32),
             weights.reshape(E), params.reshape(E))
    return out.reshape(B, N, OD)


# speedup vs baseline: 5.7145x; 1.0015x over previous
import functools

import jax
import jax.numpy as jnp
from jax import lax
from jax.experimental import pallas as pl
from jax.experimental.pallas import tpu as pltpu
from jax.experimental.pallas import tpu_sc as plsc

_NC = 2
_NS = 16
_NW = _NC * _NS
_LANES = 16


def _film_body(od, nf_ref, cond_ref, Wc_ref, bc_ref, Wf_ref, bf_ref, tbl_ref):
    nf = nf_ref[0]
    x = lax.dot_general(nf, Wf_ref[...], (((1,), (1,)), ((), ())),
                        preferred_element_type=jnp.float32)
    x = x + bf_ref[...]
    mu = jnp.mean(x, axis=1, keepdims=True)
    xc = x - mu
    var = jnp.mean(xc * xc, axis=1, keepdims=True)
    xn = xc / jnp.sqrt(var + 1e-5)
    gb = lax.dot_general(cond_ref[0], Wc_ref[...], (((1,), (1,)), ((), ())),
                         preferred_element_type=jnp.float32)
    gb = gb + bc_ref[...]
    gamma = gb[:, :od] + 1.0
    beta = gb[:, od:]
    tbl_ref[...] = jnp.maximum(gamma * xn + beta, 0.0)


def _film_call(node_feats, cond_feats, W_cond, b_cond, W_film, b_film):
    B, N, D = node_feats.shape
    OD = W_film.shape[0]
    CD = W_cond.shape[1]
    return pl.pallas_call(
        functools.partial(_film_body, OD),
        grid=(B,),
        in_specs=[
            pl.BlockSpec((1, N, D), lambda b: (b, 0, 0)),
            pl.BlockSpec((1, 1, CD), lambda b: (b, 0, 0)),
            pl.BlockSpec((2 * OD, CD), lambda b: (0, 0)),
            pl.BlockSpec((1, 2 * OD), lambda b: (0, 0)),
            pl.BlockSpec((OD, D), lambda b: (0, 0)),
            pl.BlockSpec((1, OD), lambda b: (0, 0)),
        ],
        out_specs=pl.BlockSpec((N, OD), lambda b: (b, 0)),
        out_shape=jax.ShapeDtypeStruct((B * N, OD), jnp.float32),
    )(node_feats, cond_feats, W_cond, b_cond.reshape(1, 2 * OD), W_film,
      b_film.reshape(1, OD))


def _make_sc_gather(n_nodes, OD, K, E):
    CE = 128
    CN = CE // K
    NCH = OD // _LANES
    n_chunks = E // CE
    assert n_chunks * CE == E
    BASE = n_chunks // _NW
    assert BASE % 3 == 0
    assert BASE % 2 == 0
    REM = n_chunks - _NW * BASE
    assert REM % 2 == 0
    NXW = REM // 2
    CAP = BASE + (2 if REM else 0)
    mesh = plsc.VectorSubcoreMesh(core_axis_name="c", subcore_axis_name="s")

    @functools.partial(
        pl.kernel,
        out_type=jax.ShapeDtypeStruct((n_nodes, OD), jnp.float32),
        mesh=mesh,
        scratch_types=[
            pltpu.VMEM((CAP * CE,), jnp.int32),
            pltpu.VMEM((CAP * CE,), jnp.float32),
            pltpu.VMEM((CAP * CE,), jnp.float32),
            pltpu.VMEM((3, CE, OD), jnp.float32),
            pltpu.VMEM((CAP * CN, OD), jnp.float32),
            pltpu.SemaphoreType.DMA((3,)),
        ],
    )
    def sc_gather(tbl_hbm, idx_hbm, w_hbm, p_hbm, out_hbm, idx_v, w_v, p_v,
                  rows_v, out_v, sem):
        wid = lax.axis_index("s") * _NC + lax.axis_index("c")
        start = wid * BASE + 2 * jnp.minimum(wid, NXW)
        ebase = start * CE
        has_extra = wid < NXW
        pltpu.sync_copy(idx_hbm.at[pl.ds(ebase, BASE * CE)],
                        idx_v.at[pl.ds(0, BASE * CE)])
        pltpu.sync_copy(w_hbm.at[pl.ds(ebase, BASE * CE)],
                        w_v.at[pl.ds(0, BASE * CE)])
        pltpu.sync_copy(p_hbm.at[pl.ds(ebase, BASE * CE)],
                        p_v.at[pl.ds(0, BASE * CE)])

        @pl.when(has_extra)
        def _():
            eb2 = ebase + BASE * CE
            pltpu.sync_copy(idx_hbm.at[pl.ds(eb2, 2 * CE)],
                            idx_v.at[pl.ds(BASE * CE, 2 * CE)])
            pltpu.sync_copy(w_hbm.at[pl.ds(eb2, 2 * CE)],
                            w_v.at[pl.ds(BASE * CE, 2 * CE)])
            pltpu.sync_copy(p_hbm.at[pl.ds(eb2, 2 * CE)],
                            p_v.at[pl.ds(BASE * CE, 2 * CE)])

        lane_splat = [jnp.full((_LANES,), j, jnp.int32) for j in range(_LANES)]

        def issue(ci, slot):
            pltpu.async_copy(tbl_hbm.at[idx_v.at[pl.ds(ci * CE, CE)]],
                             rows_v.at[slot], sem.at[slot])

        def wait(slot):
            pltpu.make_async_copy(tbl_hbm.at[idx_v.at[pl.ds(0, CE)]],
                                  rows_v.at[slot], sem.at[slot]).wait()

        def compute(ci, slot):
            for q in range(CN):
                acc = [jnp.zeros((_LANES,), jnp.float32) for _ in range(NCH)]
                for g in range(K // _LANES):
                    off = ci * CE + q * K + g * _LANES
                    ew = (w_v[pl.ds(off, _LANES)] * p_v[pl.ds(off, _LANES)])
                    for jj in range(_LANES):
                        e = q * K + g * _LANES + jj
                        wb = ew.at[lane_splat[jj]].get(
                            mode='promise_in_bounds')
                        for c in range(NCH):
                            r = rows_v[slot, e, pl.ds(c * _LANES, _LANES)]
                            acc[c] = acc[c] + wb * r
                row = ci * CN + q
                for c in range(NCH):
                    out_v[row, pl.ds(c * _LANES, _LANES)] = jnp.maximum(
                        acc[c], 0.0)

        n_mine = BASE + jnp.where(has_extra, 2, 0)
        issue(0, 0)
        issue(1, 1)

        def gbody(i, carry):
            slot = lax.rem(i, 3)
            wait(slot)
            compute(i, slot)
            nci = i + 2

            @pl.when(nci < n_mine)
            def _():
                issue(nci, lax.rem(nci, 3))
            return carry

        lax.fori_loop(0, n_mine, gbody, 0)

        pltpu.sync_copy(out_v.at[pl.ds(0, BASE * CN)],
                        out_hbm.at[pl.ds(start * CN, BASE * CN)])

        @pl.when(has_extra)
        def _():
            pltpu.sync_copy(
                out_v.at[pl.ds(BASE * CN, 2 * CN)],
                out_hbm.at[pl.ds(start * CN + BASE * CN, 2 * CN)])

    return sc_gather


def kernel(node_feats, cond_feats, weights, params, coords_j, W_cond, b_cond,
           W_film, b_film):
    B, N, D = node_feats.shape
    K = weights.shape[2]
    OD = W_film.shape[0]
    E = B * N * K

    tbl = _film_call(node_feats, cond_feats, W_cond, b_cond, W_film, b_film)
    sc = _make_sc_gather(B * N, OD, K, E)
    idx = (coords_j if coords_j.dtype == jnp.int32
           else coords_j.astype(jnp.int32))
    out = sc(tbl, idx, weights.reshape(E), params.reshape(E))
    return out.reshape(B, N, OD)
